# baseline (speedup 1.0000x reference)
import functools

import jax
import jax.numpy as jnp
from jax import lax
from jax.experimental import pallas as pl
from jax.experimental.pallas import tpu as pltpu
from jax.experimental.pallas import tpu_sc as plsc

NC = 2
NS = 16
NW = NC * NS

BLKB = 128
HP = 25
NBLK = 4
ROWS = 256
GBYTES = ROWS * 128 * 4
OBYTES = 2 * 32 * BLKB * 4


def _body(idx_hbm, table4_hbm, out3_hbm,
          idx_all, gidx0, gidx1, rbuf0, rbuf1, rows0, rows1, ov0, ov1,
          gsem, osem):
    wid = lax.axis_index("s") * NC + lax.axis_index("c")
    iota = lax.broadcasted_iota(jnp.int32, (16,), 0)

    def build(i, gidx, rbuf):
        h = 2 * i
        for half in range(2):
            for k in range(8):
                addr = (h + half) + 800 * k + 50 * iota
                v = plsc.load_gather(idx_all, [addr])
                gidx[pl.ds(128 * half + 16 * k, 16)] = v >> 2
                rbuf[pl.ds(128 * half + 16 * k, 16)] = (v & 3) * 32

    def transpose_half(rows_v, rbuf, out_v, half):
        def kstep(k, c):
            base = 128 * half + 16 * k
            rvec = base + iota
            rvals = rbuf[pl.ds(base, 16)]
            for f in range(32):
                vals = plsc.load_gather(rows_v, [rvec, rvals + f])
                out_v[f, pl.ds(16 * k, 16)] = vals
            return c
        lax.fori_loop(0, 8, kstep, 0, unroll=2)

    def process(rows_v, rbuf, h, b0):
        transpose_half(rows_v, rbuf, ov0, 0)
        pltpu.async_copy(ov0, out3_hbm.at[h, :, pl.ds(b0, BLKB)], osem)
        transpose_half(rows_v, rbuf, ov1, 1)
        pltpu.async_copy(ov1, out3_hbm.at[h + 1, :, pl.ds(b0, BLKB)], osem)

    def block(bi, carry):
        b0 = pl.multiple_of((4 * wid + bi) * BLKB, BLKB)
        pltpu.sync_copy(idx_hbm.at[pl.ds(b0 * 50, 50 * BLKB)], idx_all)

        def issue(gidx, rows_v):
            for q in range(4):
                pltpu.async_copy(
                    table4_hbm.at[gidx.at[pl.ds(64 * q, 64)]],
                    rows_v.at[pl.ds(64 * q, 64), :], gsem)

        build(0, gidx0, rbuf0)
        issue(gidx0, rows0)

        def wait_gather(i):
            pltpu.make_async_copy(
                table4_hbm.at[pl.ds(0, ROWS)], rows0, gsem).wait()

        def wait_out(i):
            pltpu.make_async_copy(
                ov0, out3_hbm.at[0, :, pl.ds(b0, BLKB)], osem).wait()
            pltpu.make_async_copy(
                ov1, out3_hbm.at[0, :, pl.ds(b0, BLKB)], osem).wait()

        def unit(i, carry2):
            p = lax.rem(i, 2)

            @pl.when(i < HP - 1)
            def _():
                @pl.when(p == 0)
                def _():
                    build(i + 1, gidx1, rbuf1)
                    issue(gidx1, rows1)

                @pl.when(p == 1)
                def _():
                    build(i + 1, gidx0, rbuf0)
                    issue(gidx0, rows0)

            wait_gather(i)

            @pl.when(i >= 1)
            def _():
                wait_out(i)

            h = 2 * i

            @pl.when(p == 0)
            def _():
                process(rows0, rbuf0, h, b0)

            @pl.when(p == 1)
            def _():
                process(rows1, rbuf1, h, b0)

            return carry2

        lax.fori_loop(0, HP, unit, 0)
        wait_out(HP)
        return carry

    lax.fori_loop(0, NBLK, block, 0)


@functools.partial(jax.jit, static_argnames=("n", "d"))
def _gather(flat_idx, table4, n, d):
    mesh = plsc.VectorSubcoreMesh(core_axis_name="c", subcore_axis_name="s")
    return pl.kernel(
        _body,
        out_type=jax.ShapeDtypeStruct((50, d, n // 50), jnp.float32),
        mesh=mesh,
        scratch_types=[
            pltpu.VMEM((50 * BLKB,), jnp.int32),
            pltpu.VMEM((ROWS,), jnp.int32),
            pltpu.VMEM((ROWS,), jnp.int32),
            pltpu.VMEM((ROWS,), jnp.int32),
            pltpu.VMEM((ROWS,), jnp.int32),
            pltpu.VMEM((ROWS, 128), jnp.float32),
            pltpu.VMEM((ROWS, 128), jnp.float32),
            pltpu.VMEM((32, BLKB), jnp.float32),
            pltpu.VMEM((32, BLKB), jnp.float32),
            pltpu.SemaphoreType.DMA,
            pltpu.SemaphoreType.DMA,
        ],
        compiler_params=pltpu.CompilerParams(use_tc_tiling_on_sc=True, needs_layout_passes=False),
    )(flat_idx, table4)


def kernel(action_idx, table):
    b, h = action_idx.shape
    n = b * h
    d = table.shape[1]
    flat_idx = action_idx.reshape(n).astype(jnp.int32)
    table4 = table.reshape(table.shape[0] // 4, 128)
    out3 = _gather(flat_idx, table4, n, d)
    return jnp.transpose(out3, (2, 0, 1))
